# Initial kernel scaffold; baseline (speedup 1.0000x reference)
import jax
import jax.numpy as jnp
from jax.experimental import pallas as pl


def kernel(x, emb_table, fc_w, fc_b):
    raise NotImplementedError("write your pallas kernel here")


# speedup vs baseline: 2.6167x; 2.6167x over previous
import functools

import jax
import jax.numpy as jnp
from jax import lax
from jax.experimental import pallas as pl
from jax.experimental.pallas import tpu as pltpu
from jax.experimental.pallas import tpu_sc as plsc

_NC = 2
_NS = 16
_NW = _NC * _NS

_B = 4096
_L = 50
_D = 128
_C = 32
_ROWS = _B // _NW
_VREGS = _D // 16


def _pool_body(x_ref, tab_ref, out_ref, idx_v, buf0, buf1, stage_v, sem0, sem1):
    wid = lax.axis_index("s") * _NC + lax.axis_index("c")
    base = wid * _ROWS

    pltpu.sync_copy(x_ref.at[pl.ds(base, _ROWS)], idx_v)

    pltpu.async_copy(tab_ref.at[idx_v.at[0]], buf0, sem0)
    pltpu.async_copy(tab_ref.at[idx_v.at[1]], buf1, sem1)

    def accum_row(buf, r):
        acc = [buf[0, pl.ds(16 * k, 16)] for k in range(_VREGS)]
        for j in range(1, _L):
            for k in range(_VREGS):
                acc[k] = acc[k] + buf[j, pl.ds(16 * k, 16)]
        for k in range(_VREGS):
            stage_v[r, pl.ds(16 * k, 16)] = acc[k]

    def body_fixed(g, carry):
        r0 = 2 * g
        pltpu.make_async_copy(tab_ref.at[idx_v.at[0]], buf0, sem0).wait()
        accum_row(buf0, r0)
        nxt0 = lax.min(r0 + 2, _ROWS - 2)
        pltpu.async_copy(tab_ref.at[idx_v.at[nxt0]], buf0, sem0)

        pltpu.make_async_copy(tab_ref.at[idx_v.at[1]], buf1, sem1).wait()
        accum_row(buf1, r0 + 1)
        nxt1 = lax.min(r0 + 3, _ROWS - 1)
        pltpu.async_copy(tab_ref.at[idx_v.at[nxt1]], buf1, sem1)
        return carry

    lax.fori_loop(0, _ROWS // 2, body_fixed, 0)

    pltpu.make_async_copy(tab_ref.at[idx_v.at[0]], buf0, sem0).wait()
    pltpu.make_async_copy(tab_ref.at[idx_v.at[1]], buf1, sem1).wait()

    pltpu.sync_copy(stage_v, out_ref.at[pl.ds(base, _ROWS)])


@functools.partial(
    pl.kernel,
    out_type=jax.ShapeDtypeStruct((_B, _D), jnp.float32),
    mesh=plsc.VectorSubcoreMesh(core_axis_name="c", subcore_axis_name="s"),
    scratch_types=[
        pltpu.VMEM((_ROWS, _L), jnp.int32),
        pltpu.VMEM((_L, _D), jnp.float32),
        pltpu.VMEM((_L, _D), jnp.float32),
        pltpu.VMEM((_ROWS, _D), jnp.float32),
        pltpu.SemaphoreType.DMA,
        pltpu.SemaphoreType.DMA,
    ],
)
def _pool(x_ref, tab_ref, out_ref, idx_v, buf0, buf1, stage_v, sem0, sem1):
    _pool_body(x_ref, tab_ref, out_ref, idx_v, buf0, buf1, stage_v, sem0, sem1)


def _fc_body(m_ref, w_ref, b_ref, o_ref):
    o_ref[...] = (
        jnp.dot(m_ref[...] * (1.0 / _L), w_ref[...],
                preferred_element_type=jnp.float32)
        + b_ref[...]
    )


def _fc(pooled, wt, b2):
    blk = 1024
    return pl.pallas_call(
        _fc_body,
        grid=(_B // blk,),
        in_specs=[
            pl.BlockSpec((blk, _D), lambda i: (i, 0)),
            pl.BlockSpec((_D, _C), lambda i: (0, 0)),
            pl.BlockSpec((1, _C), lambda i: (0, 0)),
        ],
        out_specs=pl.BlockSpec((blk, _C), lambda i: (i, 0)),
        out_shape=jax.ShapeDtypeStruct((_B, _C), jnp.float32),
    )(pooled, wt, b2)


@jax.jit
def kernel(x, emb_table, fc_w, fc_b):
    pooled = _pool(x, emb_table)
    wt = fc_w.T
    b2 = fc_b.reshape(1, _C)
    return _fc(pooled, wt, b2)


# speedup vs baseline: 3.6378x; 1.3902x over previous
import functools

import jax
import jax.numpy as jnp
import numpy as np
from jax import lax
from jax.experimental import pallas as pl
from jax.experimental.pallas import tpu as pltpu
from jax.experimental.pallas import tpu_sc as plsc

_NC = 2
_NS = 16
_NW = _NC * _NS

_B = 4096
_L = 50
_D = 128
_C = 32
_ROWS = _B // _NW
_CL = 128
_NCHUNK = _ROWS * _L // _CL
_ACC_ROWS = _NS * _ROWS
_NBUF = 4

_DST_TABLE = (
    (np.arange(_NS * _NCHUNK * _CL, dtype=np.int32) // _L) % _ACC_ROWS
).reshape(_NS, _NCHUNK, _CL)


def _pool_body(x_ref, dst_ref, tab_ref, out_ref, idx_v, dst_v, gbufs,
               acc_ref, gsems, ssems):
    c = lax.axis_index("c")
    s = lax.axis_index("s")
    wid = c * _NS + s
    gbase = wid * _ROWS
    lbase = s * _ROWS

    pltpu.sync_copy(x_ref.at[wid], idx_v)
    pltpu.sync_copy(dst_ref.at[s], dst_v)

    zero = jnp.zeros((16,), jnp.float32)

    def gen_zero(r, carry):
        for k in range(_D // 16):
            gbufs[0][r, pl.ds(16 * k, 16)] = zero
        return carry

    lax.fori_loop(0, _CL, gen_zero, 0)
    pltpu.sync_copy(gbufs[0], acc_ref.at[pl.ds(lbase, _ROWS)])

    def gather_chunk(j, slot):
        pltpu.async_copy(tab_ref.at[idx_v.at[j]], gbufs[slot], gsems[slot])

    def scatter_chunk(j, slot):
        pltpu.async_copy(gbufs[slot], acc_ref.at[dst_v.at[j]], ssems[slot],
                         add=True)

    def wait_gather(slot):
        pltpu.make_async_copy(tab_ref.at[idx_v.at[0]], gbufs[slot],
                              gsems[slot]).wait()

    def wait_scatter(slot):
        pltpu.make_async_copy(gbufs[slot], acc_ref.at[pl.ds(0, _CL)],
                              ssems[slot]).wait()

    for j in range(2):
        gather_chunk(j, j)
    for j in range(2):
        wait_gather(j)
        scatter_chunk(j, j)
        gather_chunk(j + 2, j + 2)

    def body4(g, carry):
        j0 = 2 + _NBUF * g
        for q in range(_NBUF):
            j = j0 + q
            p = (2 + q) % _NBUF
            wait_gather(p)
            scatter_chunk(j, p)
            nslot = (p + 2) % _NBUF

            @pl.when(j + 2 <= _NCHUNK - 1)
            def _():
                wait_scatter(nslot)
                gather_chunk(j + 2, nslot)

        return carry

    lax.fori_loop(0, (_NCHUNK - 2) // _NBUF, body4, 0)

    for p in range(_NBUF):
        wait_scatter(p)
    pltpu.sync_copy(acc_ref.at[pl.ds(lbase, _ROWS)],
                    out_ref.at[pl.ds(gbase, _ROWS)])


@functools.partial(
    pl.kernel,
    out_type=jax.ShapeDtypeStruct((_B, _D), jnp.float32),
    mesh=plsc.VectorSubcoreMesh(core_axis_name="c", subcore_axis_name="s"),
    scratch_types=[
        pltpu.VMEM((_NCHUNK, _CL), jnp.int32),
        pltpu.VMEM((_NCHUNK, _CL), jnp.int32),
        [pltpu.VMEM((_CL, _D), jnp.float32) for _ in range(_NBUF)],
        pltpu.VMEM_SHARED((_ACC_ROWS, _D), jnp.float32),
        [pltpu.SemaphoreType.DMA for _ in range(_NBUF)],
        [pltpu.SemaphoreType.DMA for _ in range(_NBUF)],
    ],
)
def _pool(x_ref, dst_ref, tab_ref, out_ref, idx_v, dst_v, gbufs, acc_ref,
          gsems, ssems):
    _pool_body(x_ref, dst_ref, tab_ref, out_ref, idx_v, dst_v, gbufs,
               acc_ref, gsems, ssems)


def _fc_body(m_ref, w_ref, b_ref, o_ref):
    o_ref[...] = (
        jnp.dot(m_ref[...] * (1.0 / _L), w_ref[...],
                preferred_element_type=jnp.float32)
        + b_ref[...]
    )


def _fc(pooled, wt, b2):
    blk = 1024
    return pl.pallas_call(
        _fc_body,
        grid=(_B // blk,),
        in_specs=[
            pl.BlockSpec((blk, _D), lambda i: (i, 0)),
            pl.BlockSpec((_D, _C), lambda i: (0, 0)),
            pl.BlockSpec((1, _C), lambda i: (0, 0)),
        ],
        out_specs=pl.BlockSpec((blk, _C), lambda i: (i, 0)),
        out_shape=jax.ShapeDtypeStruct((_B, _C), jnp.float32),
    )(pooled, wt, b2)


@jax.jit
def kernel(x, emb_table, fc_w, fc_b):
    xf = x.reshape(_NW, _NCHUNK, _CL)
    dst = jnp.asarray(_DST_TABLE)
    pooled = _pool(xf, dst, emb_table)
    wt = fc_w.T
    b2 = fc_b.reshape(1, _C)
    return _fc(pooled, wt, b2)
